# baseline (speedup 1.0000x reference)
import jax
import jax.numpy as jnp
from jax import lax
from jax.experimental import pallas as pl
from jax.experimental.pallas import tpu as pltpu
from jax.experimental.pallas import tpu_sc as plsc

NC = 2
NS = 16
NW = NC * NS
CH = 80
NBUF = 10


def _deg_kernel(zeros_hbm, dst_hbm, out_hbm, dstv, ones, degsh, dsem):
    c = lax.axis_index("c")
    s = lax.axis_index("s")
    wid = s * NC + c
    ept = dstv.shape[0]
    nchunk = ept // CH
    nstep = nchunk // NBUF

    @pl.when(s == 0)
    def _():
        pltpu.sync_copy(zeros_hbm.at[c], degsh)

    pltpu.sync_copy(dst_hbm.at[pl.ds(wid * ept, ept)], dstv)
    for i in range(CH // 16):
        ones[pl.ds(i * 16, 16)] = jnp.ones((16,), jnp.float32)
    plsc.subcore_barrier()

    rem = nchunk - nstep * NBUF

    def body(k, carry):
        base = k * NBUF
        for b in range(NBUF):
            idx = dstv.at[pl.ds((base + b) * CH, CH)]
            pltpu.async_copy(ones, degsh.at[idx], dsem.at[b], add=True)
        for b in range(NBUF):
            idx = dstv.at[pl.ds((base + b) * CH, CH)]
            pltpu.make_async_copy(ones, degsh.at[idx], dsem.at[b]).wait()
        return carry

    lax.fori_loop(0, nstep, body, 0)
    for b in range(rem):
        idx = dstv.at[pl.ds((nstep * NBUF + b) * CH, CH)]
        pltpu.async_copy(ones, degsh.at[idx], dsem.at[b], add=True)
    for b in range(rem):
        idx = dstv.at[pl.ds((nstep * NBUF + b) * CH, CH)]
        pltpu.make_async_copy(ones, degsh.at[idx], dsem.at[b]).wait()
    plsc.subcore_barrier()

    @pl.when(s == 0)
    def _():
        pltpu.sync_copy(degsh, out_hbm.at[c])


def _scatter_kernel(m_hbm, src_hbm, dst_hbm, out_hbm, srcv, dstv, rows, accsh,
                    gsem, ssem):
    c = lax.axis_index("c")
    s = lax.axis_index("s")
    wid = s * NC + c
    n = m_hbm.shape[0]
    ept = srcv.shape[0]
    nchunk = ept // CH
    rpt = (-(-n // NS) + 7) // 8 * 8
    tail = n - (NS - 1) * rpt
    r0 = pl.multiple_of(s * rpt, 8)

    @pl.when(s < NS - 1)
    def _():
        pltpu.sync_copy(m_hbm.at[pl.ds(r0, rpt)], accsh.at[pl.ds(r0, rpt)])

    @pl.when(s == NS - 1)
    def _():
        pltpu.sync_copy(
            m_hbm.at[pl.ds((NS - 1) * rpt, tail)],
            accsh.at[pl.ds((NS - 1) * rpt, tail)],
        )

    pltpu.sync_copy(src_hbm.at[pl.ds(wid * ept, ept)], srcv)
    pltpu.sync_copy(dst_hbm.at[pl.ds(wid * ept, ept)], dstv)
    plsc.subcore_barrier()

    def sidx(j):
        return srcv.at[pl.ds(j * CH, CH)]

    def didx(j):
        return dstv.at[pl.ds(j * CH, CH)]

    nstep = nchunk // NBUF
    rem = nchunk - nstep * NBUF
    for b in range(NBUF):
        pltpu.async_copy(m_hbm.at[sidx(b)], rows.at[b], gsem.at[b])

    def body(k, carry):
        base = k * NBUF
        for b in range(NBUF):
            j = base + b
            pltpu.make_async_copy(m_hbm.at[sidx(j)], rows.at[b], gsem.at[b]).wait()
            pltpu.async_copy(rows.at[b], accsh.at[didx(j)], ssem.at[b], add=True)

        for b in range(rem):
            j = base + b
            pltpu.make_async_copy(rows.at[b], accsh.at[didx(j)], ssem.at[b]).wait()
            pltpu.async_copy(m_hbm.at[sidx(j + NBUF)], rows.at[b], gsem.at[b])

        @pl.when(k < nstep - 1)
        def _():
            for b in range(rem, NBUF):
                j = base + b
                pltpu.make_async_copy(rows.at[b], accsh.at[didx(j)], ssem.at[b]).wait()
                pltpu.async_copy(m_hbm.at[sidx(j + NBUF)], rows.at[b], gsem.at[b])

        return carry

    lax.fori_loop(0, nstep, body, 0)
    for b in range(rem):
        j = nstep * NBUF + b
        pltpu.make_async_copy(m_hbm.at[sidx(j)], rows.at[b], gsem.at[b]).wait()
        pltpu.async_copy(rows.at[b], accsh.at[didx(j)], ssem.at[b], add=True)
    for b in range(NBUF):
        pltpu.make_async_copy(rows.at[b], accsh.at[didx(b)], ssem.at[b]).wait()
    plsc.subcore_barrier()

    @pl.when(s < NS - 1)
    def _():
        pltpu.sync_copy(accsh.at[pl.ds(r0, rpt)], out_hbm.at[c, pl.ds(r0, rpt)])

    @pl.when(s == NS - 1)
    def _():
        pltpu.sync_copy(
            accsh.at[pl.ds((NS - 1) * rpt, tail)],
            out_hbm.at[c, pl.ds((NS - 1) * rpt, tail)],
        )


def _lin1_kernel(part_ref, x_ref, w_ref, m_ref):
    deg = part_ref[0] + part_ref[1] + 1.0
    dinv = lax.rsqrt(deg)
    h = jnp.dot(x_ref[...], w_ref[...], preferred_element_type=jnp.float32)
    m_ref[...] = h * dinv[:, None]


def _lin2_kernel(part_ref, acc_ref, m1_ref, b1_ref, w2_ref, m2_ref):
    deg = part_ref[0] + part_ref[1] + 1.0
    dinv = lax.rsqrt(deg)[:, None]
    a = acc_ref[0] + acc_ref[1] - m1_ref[...]
    h = jnp.maximum(a * dinv + b1_ref[...], 0.0)
    m2_ref[...] = jnp.dot(h, w2_ref[...], preferred_element_type=jnp.float32) * dinv


def _out_kernel(part_ref, acc_ref, m2_ref, b2_ref, z_ref):
    deg = part_ref[0] + part_ref[1] + 1.0
    dinv = lax.rsqrt(deg)[:, None]
    z_ref[...] = (acc_ref[0] + acc_ref[1] - m2_ref[...]) * dinv + b2_ref[...]


def _make_scatter(n, d, ept, mesh):
    return pl.kernel(
        _scatter_kernel,
        out_type=jax.ShapeDtypeStruct((NC, n, d), jnp.float32),
        mesh=mesh,
        compiler_params=pltpu.CompilerParams(use_tc_tiling_on_sc=False),
        scratch_types=[
            pltpu.VMEM((ept,), jnp.int32),
            pltpu.VMEM((ept,), jnp.int32),
            pltpu.VMEM((NBUF, CH, d), jnp.float32),
            pltpu.VMEM_SHARED((n, d), jnp.float32),
            pltpu.SemaphoreType.DMA((NBUF,)),
            pltpu.SemaphoreType.DMA((NBUF,)),
        ],
    )


def kernel(x, ei, W1, b1, W2, b2):
    n, _ = x.shape
    e = ei.shape[1]
    d1 = W1.shape[1]
    d2 = W2.shape[1]
    ept = e // NW
    ei = ei.astype(jnp.int32)
    dst = ei[1]
    src = lax.optimization_barrier(ei)[0]
    zeros2 = jnp.zeros((NC, n), jnp.float32)

    mesh = plsc.VectorSubcoreMesh(
        core_axis_name="c", subcore_axis_name="s", num_cores=NC, num_subcores=NS
    )

    deg_fn = pl.kernel(
        _deg_kernel,
        out_type=jax.ShapeDtypeStruct((NC, n), jnp.float32),
        mesh=mesh,
        compiler_params=pltpu.CompilerParams(use_tc_tiling_on_sc=False),
        scratch_types=[
            pltpu.VMEM((ept,), jnp.int32),
            pltpu.VMEM((CH,), jnp.float32),
            pltpu.VMEM_SHARED((n,), jnp.float32),
            pltpu.SemaphoreType.DMA((NBUF,)),
        ],
    )
    part = deg_fn(zeros2, dst)

    m1 = pl.pallas_call(
        _lin1_kernel, out_shape=jax.ShapeDtypeStruct((n, d1), jnp.float32)
    )(part, x, W1)

    acc1 = _make_scatter(n, d1, ept, mesh)(m1, src, dst)

    m2 = pl.pallas_call(
        _lin2_kernel, out_shape=jax.ShapeDtypeStruct((n, d2), jnp.float32)
    )(part, acc1, m1, b1.reshape(1, d1), W2)

    acc2 = _make_scatter(n, d2, ept, mesh)(m2, src, dst)

    z = pl.pallas_call(
        _out_kernel, out_shape=jax.ShapeDtypeStruct((n, d2), jnp.float32)
    )(part, acc2, m2, b2.reshape(1, d2))
    return z


# speedup vs baseline: 1.0723x; 1.0723x over previous
import jax
import jax.numpy as jnp
from jax import lax
from jax.experimental import pallas as pl
from jax.experimental.pallas import tpu as pltpu
from jax.experimental.pallas import tpu_sc as plsc

NC = 2
NS = 16
NW = NC * NS
CH = 80
NBUF = 15


def _deg_kernel(zeros_hbm, dst_hbm, out_hbm, dstv, ones, degsh, dsem):
    c = lax.axis_index("c")
    s = lax.axis_index("s")
    wid = s * NC + c
    ept = dstv.shape[0]
    nchunk = ept // CH
    nstep = nchunk // NBUF

    @pl.when(s == 0)
    def _():
        pltpu.sync_copy(zeros_hbm.at[c], degsh)

    pltpu.sync_copy(dst_hbm.at[pl.ds(wid * ept, ept)], dstv)
    for i in range(CH // 16):
        ones[pl.ds(i * 16, 16)] = jnp.ones((16,), jnp.float32)
    plsc.subcore_barrier()

    rem = nchunk - nstep * NBUF

    def body(k, carry):
        base = k * NBUF
        for b in range(NBUF):
            idx = dstv.at[pl.ds((base + b) * CH, CH)]
            pltpu.async_copy(ones, degsh.at[idx], dsem.at[b], add=True)
        for b in range(NBUF):
            idx = dstv.at[pl.ds((base + b) * CH, CH)]
            pltpu.make_async_copy(ones, degsh.at[idx], dsem.at[b]).wait()
        return carry

    lax.fori_loop(0, nstep, body, 0)
    for b in range(rem):
        idx = dstv.at[pl.ds((nstep * NBUF + b) * CH, CH)]
        pltpu.async_copy(ones, degsh.at[idx], dsem.at[b], add=True)
    for b in range(rem):
        idx = dstv.at[pl.ds((nstep * NBUF + b) * CH, CH)]
        pltpu.make_async_copy(ones, degsh.at[idx], dsem.at[b]).wait()
    plsc.subcore_barrier()

    @pl.when(s == 0)
    def _():
        pltpu.sync_copy(degsh, out_hbm.at[c])


def _scatter_kernel(m_hbm, src_hbm, dst_hbm, out_hbm, srcv, dstv, rows, accsh,
                    gsem, ssem):
    c = lax.axis_index("c")
    s = lax.axis_index("s")
    wid = s * NC + c
    n = m_hbm.shape[0]
    ept = srcv.shape[0]
    nchunk = ept // CH
    rpt = (-(-n // NS) + 7) // 8 * 8
    tail = n - (NS - 1) * rpt
    r0 = pl.multiple_of(s * rpt, 8)

    @pl.when(s < NS - 1)
    def _():
        pltpu.sync_copy(m_hbm.at[pl.ds(r0, rpt)], accsh.at[pl.ds(r0, rpt)])

    @pl.when(s == NS - 1)
    def _():
        pltpu.sync_copy(
            m_hbm.at[pl.ds((NS - 1) * rpt, tail)],
            accsh.at[pl.ds((NS - 1) * rpt, tail)],
        )

    pltpu.sync_copy(src_hbm.at[pl.ds(wid * ept, ept)], srcv)
    pltpu.sync_copy(dst_hbm.at[pl.ds(wid * ept, ept)], dstv)
    plsc.subcore_barrier()

    def sidx(j):
        return srcv.at[pl.ds(j * CH, CH)]

    def didx(j):
        return dstv.at[pl.ds(j * CH, CH)]

    nstep = nchunk // NBUF
    rem = nchunk - nstep * NBUF
    for b in range(NBUF):
        pltpu.async_copy(m_hbm.at[sidx(b)], rows.at[b], gsem.at[b])

    def body(k, carry):
        base = k * NBUF
        for b in range(NBUF):
            j = base + b
            pltpu.make_async_copy(m_hbm.at[sidx(j)], rows.at[b], gsem.at[b]).wait()
            pltpu.async_copy(rows.at[b], accsh.at[didx(j)], ssem.at[b], add=True)

        for b in range(rem):
            j = base + b
            pltpu.make_async_copy(rows.at[b], accsh.at[didx(j)], ssem.at[b]).wait()
            pltpu.async_copy(m_hbm.at[sidx(j + NBUF)], rows.at[b], gsem.at[b])

        @pl.when(k < nstep - 1)
        def _():
            for b in range(rem, NBUF):
                j = base + b
                pltpu.make_async_copy(rows.at[b], accsh.at[didx(j)], ssem.at[b]).wait()
                pltpu.async_copy(m_hbm.at[sidx(j + NBUF)], rows.at[b], gsem.at[b])

        return carry

    lax.fori_loop(0, nstep, body, 0)
    for b in range(rem):
        j = nstep * NBUF + b
        pltpu.make_async_copy(m_hbm.at[sidx(j)], rows.at[b], gsem.at[b]).wait()
        pltpu.async_copy(rows.at[b], accsh.at[didx(j)], ssem.at[b], add=True)
    for b in range(NBUF):
        pltpu.make_async_copy(rows.at[b], accsh.at[didx(b)], ssem.at[b]).wait()
    plsc.subcore_barrier()

    @pl.when(s < NS - 1)
    def _():
        pltpu.sync_copy(accsh.at[pl.ds(r0, rpt)], out_hbm.at[c, pl.ds(r0, rpt)])

    @pl.when(s == NS - 1)
    def _():
        pltpu.sync_copy(
            accsh.at[pl.ds((NS - 1) * rpt, tail)],
            out_hbm.at[c, pl.ds((NS - 1) * rpt, tail)],
        )


def _lin1_kernel(part_ref, x_ref, w_ref, m_ref):
    deg = part_ref[0] + part_ref[1] + 1.0
    dinv = lax.rsqrt(deg)
    h = jnp.dot(x_ref[...], w_ref[...], preferred_element_type=jnp.float32)
    m_ref[...] = h * dinv[:, None]


def _lin2_kernel(part_ref, acc_ref, m1_ref, b1_ref, w2_ref, m2_ref):
    deg = part_ref[0] + part_ref[1] + 1.0
    dinv = lax.rsqrt(deg)[:, None]
    a = acc_ref[0] + acc_ref[1] - m1_ref[...]
    h = jnp.maximum(a * dinv + b1_ref[...], 0.0)
    m2_ref[...] = jnp.dot(h, w2_ref[...], preferred_element_type=jnp.float32) * dinv


def _out_kernel(part_ref, acc_ref, m2_ref, b2_ref, z_ref):
    deg = part_ref[0] + part_ref[1] + 1.0
    dinv = lax.rsqrt(deg)[:, None]
    z_ref[...] = (acc_ref[0] + acc_ref[1] - m2_ref[...]) * dinv + b2_ref[...]


def _make_scatter(n, d, ept, mesh):
    return pl.kernel(
        _scatter_kernel,
        out_type=jax.ShapeDtypeStruct((NC, n, d), jnp.float32),
        mesh=mesh,
        compiler_params=pltpu.CompilerParams(use_tc_tiling_on_sc=False),
        scratch_types=[
            pltpu.VMEM((ept,), jnp.int32),
            pltpu.VMEM((ept,), jnp.int32),
            pltpu.VMEM((NBUF, CH, d), jnp.float32),
            pltpu.VMEM_SHARED((n, d), jnp.float32),
            pltpu.SemaphoreType.DMA((NBUF,)),
            pltpu.SemaphoreType.DMA((NBUF,)),
        ],
    )


def kernel(x, ei, W1, b1, W2, b2):
    n, _ = x.shape
    e = ei.shape[1]
    d1 = W1.shape[1]
    d2 = W2.shape[1]
    ept = e // NW
    ei = ei.astype(jnp.int32)
    src = ei[0]
    dst = ei[1]
    zeros2 = jnp.zeros((NC, n), jnp.float32)

    mesh = plsc.VectorSubcoreMesh(
        core_axis_name="c", subcore_axis_name="s", num_cores=NC, num_subcores=NS
    )

    deg_fn = pl.kernel(
        _deg_kernel,
        out_type=jax.ShapeDtypeStruct((NC, n), jnp.float32),
        mesh=mesh,
        compiler_params=pltpu.CompilerParams(use_tc_tiling_on_sc=False),
        scratch_types=[
            pltpu.VMEM((ept,), jnp.int32),
            pltpu.VMEM((CH,), jnp.float32),
            pltpu.VMEM_SHARED((n,), jnp.float32),
            pltpu.SemaphoreType.DMA((NBUF,)),
        ],
    )
    part = deg_fn(zeros2, dst)

    m1 = pl.pallas_call(
        _lin1_kernel, out_shape=jax.ShapeDtypeStruct((n, d1), jnp.float32)
    )(part, x, W1)

    acc1 = _make_scatter(n, d1, ept, mesh)(m1, src, dst)

    m2 = pl.pallas_call(
        _lin2_kernel, out_shape=jax.ShapeDtypeStruct((n, d2), jnp.float32)
    )(part, acc1, m1, b1.reshape(1, d1), W2)

    acc2 = _make_scatter(n, d2, ept, mesh)(m2, src, dst)

    z = pl.pallas_call(
        _out_kernel, out_shape=jax.ShapeDtypeStruct((n, d2), jnp.float32)
    )(part, acc2, m2, b2.reshape(1, d2))
    return z
